# baseline (speedup 1.0000x reference)
import functools

import jax
import jax.numpy as jnp
from jax import lax
from jax.experimental import pallas as pl
from jax.experimental.pallas import tpu as pltpu, tpu_sc as plsc

B = 16384
UD = 64
UDP = 128
SD = 32
NT = 16
OUT_D = UD + 3 * SD


def _sc_gather_body(table_hbm, idx_hbm, out_hbm, idx_v, rows_v, sem,
                    *, n_chunks, chunk, b_per_w, nc):
    wid = lax.axis_index("s") * nc + lax.axis_index("c")
    base = wid * b_per_w
    pltpu.sync_copy(idx_hbm.at[pl.ds(base, b_per_w)], idx_v)
    copies = [
        pltpu.async_copy(table_hbm.at[idx_v.at[pl.ds(j * chunk, chunk)]],
                         rows_v.at[pl.ds(j * chunk, chunk)], sem)
        for j in range(n_chunks)
    ]
    for c in copies:
        c.wait()
    pltpu.sync_copy(rows_v, out_hbm.at[pl.ds(base, b_per_w)])


def _unit_gather_sc(unit_table, unit_ids):
    info = plsc.get_sparse_core_info()
    nc, ns = info.num_cores, info.num_subcores
    nw = nc * ns
    b_per_w = B // nw
    chunk = 128
    n_chunks = b_per_w // chunk
    mesh = plsc.VectorSubcoreMesh(core_axis_name="c", subcore_axis_name="s")
    kern = pl.kernel(
        functools.partial(_sc_gather_body, n_chunks=n_chunks, chunk=chunk,
                          b_per_w=b_per_w, nc=nc),
        out_type=jax.ShapeDtypeStruct((B, UDP), jnp.float32),
        mesh=mesh,
        scratch_types=[
            pltpu.VMEM((b_per_w,), jnp.int32),
            pltpu.VMEM((b_per_w, UDP), jnp.float32),
            pltpu.SemaphoreType.DMA,
        ],
        compiler_params=pltpu.CompilerParams(use_tc_tiling_on_sc=True),
    )
    return kern(jnp.pad(unit_table, ((0, 0), (0, UDP - UD))), unit_ids)


NSLOT = 20
KL = NSLOT * NT


def _ext_block(table, query, nrows, t):
    s = jnp.sum(table * query, axis=1, keepdims=True)
    m = jnp.max(s[:nrows, :])
    w = jnp.exp(s - m)
    z = lambda c: jnp.zeros((NT, c), jnp.float32)
    parts = [(SD * t, None), (0, table * w), (2 * SD - SD * t, None),
             (t, None), (0, w), (3 - t, None)]
    return jnp.concatenate(
        [p if p is not None else z(c) for c, p in parts if p is not None or c],
        axis=1)


def _tc_pool_body(aid_ref, tid_ref, sid_ref,
                  at_ref, tt_ref, st_ref, aq_ref, tq_ref, sq_ref, out_ref):

    extall = jnp.concatenate([
        _ext_block(at_ref[...], aq_ref[...], 14, 0),
        _ext_block(tt_ref[...], tq_ref[...], 16, 1),
        _ext_block(st_ref[...], sq_ref[...], 4, 2)], axis=0)
    ri = lax.broadcasted_iota(jnp.int32, (KL, 48), 0)
    ci = lax.broadcasted_iota(jnp.int32, (KL, 48), 1)
    sel = (ci == ((ri // (8 * NT)).clip(0, 2) * NT + ri % NT))
    ext = jnp.dot(sel.astype(jnp.float32), extall,
                  preferred_element_type=jnp.float32,
                  precision=lax.Precision.HIGHEST)

    def spread_mat(nslots, slot0):
        sr = lax.broadcasted_iota(jnp.int32, (nslots, KL), 0)
        sc = lax.broadcasted_iota(jnp.int32, (nslots, KL), 1)
        return (sc // NT == sr + slot0).astype(jnp.float32)

    idr = (jnp.dot(aid_ref[...].astype(jnp.float32), spread_mat(8, 0),
                   preferred_element_type=jnp.float32)
           + jnp.dot(tid_ref[...].astype(jnp.float32), spread_mat(8, 8),
                     preferred_element_type=jnp.float32)
           + jnp.dot(sid_ref[...].astype(jnp.float32), spread_mat(4, 16),
                     preferred_element_type=jnp.float32))
    jmod = lax.broadcasted_iota(jnp.int32, (1, KL), 1) % NT
    eq = (idr == jmod.astype(jnp.float32)).astype(jnp.float32)
    nd = jnp.dot(eq, ext, preferred_element_type=jnp.float32)

    rec = 1.0 / nd[:, 96:99]
    rr = lax.broadcasted_iota(jnp.int32, (3, 3 * SD), 0)
    rc = lax.broadcasted_iota(jnp.int32, (3, 3 * SD), 1)
    rep3 = (rc // SD == rr).astype(jnp.float32)
    rec_hi = rec.astype(jnp.bfloat16).astype(jnp.float32)
    rec_lo = rec - rec_hi
    rec96 = (jnp.dot(rec_hi, rep3, preferred_element_type=jnp.float32)
             + jnp.dot(rec_lo, rep3, preferred_element_type=jnp.float32))
    out_ref[...] = nd[:, 0:3 * SD] * rec96


def _pool_tc(ability_ids, trait_ids, status_ids,
             at_p, tt_p, st_p, aq, tq, sq, *, interpret=False):
    R = 1024
    grid = (B // R,)
    row_spec = lambda w: pl.BlockSpec((R, w), lambda i: (i, 0))
    full = lambda a: pl.BlockSpec(a.shape, lambda i: (0, 0))
    return pl.pallas_call(
        _tc_pool_body,
        grid=grid,
        in_specs=[row_spec(8), row_spec(8), row_spec(4),
                  full(at_p), full(tt_p), full(st_p),
                  full(aq), full(tq), full(sq)],
        out_specs=row_spec(3 * SD),
        out_shape=jax.ShapeDtypeStruct((B, 3 * SD), jnp.float32),
        interpret=interpret,
    )(ability_ids, trait_ids, status_ids,
      at_p, tt_p, st_p, aq, tq, sq)


def _merge_body(uf_ref, p_ref, out_ref):
    out_ref[:, 0:UD] = uf_ref[:, 0:UD]
    out_ref[:, UD:OUT_D] = p_ref[...]


def _merge_tc(ufeat, pooled, *, interpret=False):
    R = 1024
    return pl.pallas_call(
        _merge_body,
        grid=(B // R,),
        in_specs=[pl.BlockSpec((R, UDP), lambda i: (i, 0)),
                  pl.BlockSpec((R, 3 * SD), lambda i: (i, 0))],
        out_specs=pl.BlockSpec((R, OUT_D), lambda i: (i, 0)),
        out_shape=jax.ShapeDtypeStruct((B, OUT_D), jnp.float32),
        interpret=interpret,
    )(ufeat, pooled)


def kernel(unit_ids, ability_ids, trait_ids, status_ids,
           unit_table, ability_table, trait_table, status_table,
           ability_query, trait_query, status_query):
    ufeat = _unit_gather_sc(unit_table, unit_ids)
    at_p = jnp.zeros((NT, SD), jnp.float32).at[:14].set(ability_table)
    tt_p = trait_table
    st_p = jnp.zeros((NT, SD), jnp.float32).at[:4].set(status_table)
    pooled = _pool_tc(ability_ids, trait_ids, status_ids,
                      at_p, tt_p, st_p,
                      ability_query.reshape(1, SD),
                      trait_query.reshape(1, SD),
                      status_query.reshape(1, SD))
    return _merge_tc(ufeat, pooled)


# speedup vs baseline: 1.1887x; 1.1887x over previous
import functools

import jax
import jax.numpy as jnp
from jax import lax
from jax.experimental import pallas as pl
from jax.experimental.pallas import tpu as pltpu, tpu_sc as plsc

B = 16384
UD = 64
UDP = 128
SD = 32
NT = 16
OUT_D = UD + 3 * SD


def _sc_gather_body(table_hbm, idx_hbm, out_hbm, idx_v, rows_v, sem,
                    *, n_chunks, chunk, b_per_w, nc):
    wid = lax.axis_index("s") * nc + lax.axis_index("c")
    base = wid * b_per_w
    pltpu.sync_copy(idx_hbm.at[pl.ds(base, b_per_w)], idx_v)
    copies = [
        pltpu.async_copy(table_hbm.at[idx_v.at[pl.ds(j * chunk, chunk)]],
                         rows_v.at[pl.ds(j * chunk, chunk)], sem)
        for j in range(n_chunks)
    ]
    for c in copies:
        c.wait()
    pltpu.sync_copy(rows_v, out_hbm.at[pl.ds(base, b_per_w)])


def _unit_gather_sc(unit_table, unit_ids):
    info = plsc.get_sparse_core_info()
    nc, ns = info.num_cores, info.num_subcores
    nw = nc * ns
    b_per_w = B // nw
    chunk = 128
    n_chunks = b_per_w // chunk
    mesh = plsc.VectorSubcoreMesh(core_axis_name="c", subcore_axis_name="s")
    kern = pl.kernel(
        functools.partial(_sc_gather_body, n_chunks=n_chunks, chunk=chunk,
                          b_per_w=b_per_w, nc=nc),
        out_type=jax.ShapeDtypeStruct((B, UDP), jnp.float32),
        mesh=mesh,
        scratch_types=[
            pltpu.VMEM((b_per_w,), jnp.int32),
            pltpu.VMEM((b_per_w, UDP), jnp.float32),
            pltpu.SemaphoreType.DMA,
        ],
        compiler_params=pltpu.CompilerParams(use_tc_tiling_on_sc=True),
    )
    return kern(jnp.pad(unit_table, ((0, 0), (0, UDP - UD))), unit_ids)


NSLOT = 20
KL = NSLOT * NT


def _ext_block(table, query, nrows, t):
    s = jnp.sum(table * query, axis=1, keepdims=True)
    m = jnp.max(s[:nrows, :])
    w = jnp.exp(s - m)
    z = lambda c: jnp.zeros((NT, c), jnp.float32)
    parts = [(SD * t, None), (0, table * w), (2 * SD - SD * t, None),
             (t, None), (0, w), (3 - t, None)]
    return jnp.concatenate(
        [p if p is not None else z(c) for c, p in parts if p is not None or c],
        axis=1)


def _tc_pool_body(uf_ref, aid_ref, tid_ref, sid_ref,
                  at_ref, tt_ref, st_ref, aq_ref, tq_ref, sq_ref, out_ref):
    R = uf_ref.shape[0]
    out_ref[:, 0:UD] = uf_ref[:, 0:UD]

    extall = jnp.concatenate([
        _ext_block(at_ref[...], aq_ref[...], 14, 0),
        _ext_block(tt_ref[...], tq_ref[...], 16, 1),
        _ext_block(st_ref[...], sq_ref[...], 4, 2)], axis=0)
    ri = lax.broadcasted_iota(jnp.int32, (KL, 48), 0)
    ci = lax.broadcasted_iota(jnp.int32, (KL, 48), 1)
    sel = (ci == ((ri // (8 * NT)).clip(0, 2) * NT + ri % NT))
    ext = jnp.dot(sel.astype(jnp.float32), extall,
                  preferred_element_type=jnp.float32,
                  precision=lax.Precision.HIGHEST)

    def spread_mat(nslots, slot0):
        sr = lax.broadcasted_iota(jnp.int32, (nslots, KL), 0)
        sc = lax.broadcasted_iota(jnp.int32, (nslots, KL), 1)
        return (sc // NT == sr + slot0).astype(jnp.float32)

    idr = (jnp.dot(aid_ref[...].astype(jnp.float32), spread_mat(8, 0),
                   preferred_element_type=jnp.float32)
           + jnp.dot(tid_ref[...].astype(jnp.float32), spread_mat(8, 8),
                     preferred_element_type=jnp.float32)
           + jnp.dot(sid_ref[...].astype(jnp.float32), spread_mat(4, 16),
                     preferred_element_type=jnp.float32))
    jmod = lax.broadcasted_iota(jnp.int32, (1, KL), 1) % NT
    eq = (idr == jmod.astype(jnp.float32)).astype(jnp.float32)
    nd = jnp.dot(eq, ext, preferred_element_type=jnp.float32)

    rec = 1.0 / nd[:, 96:99]
    rr = lax.broadcasted_iota(jnp.int32, (3, 3 * SD), 0)
    rc = lax.broadcasted_iota(jnp.int32, (3, 3 * SD), 1)
    rep3 = (rc // SD == rr).astype(jnp.float32)
    rec_hi = rec.astype(jnp.bfloat16).astype(jnp.float32)
    rec_lo = rec - rec_hi
    rec96 = (jnp.dot(rec_hi, rep3, preferred_element_type=jnp.float32)
             + jnp.dot(rec_lo, rep3, preferred_element_type=jnp.float32))
    out_ref[:, UD:OUT_D] = nd[:, 0:3 * SD] * rec96


def _pool_tc(ufeat, ability_ids, trait_ids, status_ids,
             at_p, tt_p, st_p, aq, tq, sq, *, interpret=False):
    R = 1024
    grid = (B // R,)
    row_spec = lambda w: pl.BlockSpec((R, w), lambda i: (i, 0))
    full = lambda a: pl.BlockSpec(a.shape, lambda i: (0, 0))
    return pl.pallas_call(
        _tc_pool_body,
        grid=grid,
        in_specs=[row_spec(UDP), row_spec(8), row_spec(8), row_spec(4),
                  full(at_p), full(tt_p), full(st_p),
                  full(aq), full(tq), full(sq)],
        out_specs=row_spec(OUT_D),
        out_shape=jax.ShapeDtypeStruct((B, OUT_D), jnp.float32),
        interpret=interpret,
    )(ufeat, ability_ids, trait_ids, status_ids,
      at_p, tt_p, st_p, aq, tq, sq)


def kernel(unit_ids, ability_ids, trait_ids, status_ids,
           unit_table, ability_table, trait_table, status_table,
           ability_query, trait_query, status_query):
    ufeat = _unit_gather_sc(unit_table, unit_ids)
    at_p = jnp.zeros((NT, SD), jnp.float32).at[:14].set(ability_table)
    tt_p = trait_table
    st_p = jnp.zeros((NT, SD), jnp.float32).at[:4].set(status_table)
    return _pool_tc(ufeat, ability_ids, trait_ids, status_ids,
                    at_p, tt_p, st_p,
                    ability_query.reshape(1, SD),
                    trait_query.reshape(1, SD),
                    status_query.reshape(1, SD))


# speedup vs baseline: 1.2573x; 1.0577x over previous
import functools

import jax
import jax.numpy as jnp
from jax import lax
from jax.experimental import pallas as pl
from jax.experimental.pallas import tpu as pltpu, tpu_sc as plsc

B = 16384
UD = 64
UDP = 128
SD = 32
NT = 16
OUT_D = UD + 3 * SD


def _sc_gather_body(table_hbm, idx_hbm, out_hbm, idx_v, rows_v, sem,
                    *, n_chunks, chunk, b_per_w, nc):
    wid = lax.axis_index("s") * nc + lax.axis_index("c")
    base = wid * b_per_w
    pltpu.sync_copy(idx_hbm.at[pl.ds(base, b_per_w)], idx_v)
    copies = [
        pltpu.async_copy(table_hbm.at[idx_v.at[pl.ds(j * chunk, chunk)]],
                         rows_v.at[pl.ds(j * chunk, chunk)], sem)
        for j in range(n_chunks)
    ]
    for c in copies:
        c.wait()
    pltpu.sync_copy(rows_v, out_hbm.at[pl.ds(base, b_per_w)])


def _unit_gather_sc(unit_table, unit_ids):
    info = plsc.get_sparse_core_info()
    nc, ns = info.num_cores, info.num_subcores
    nw = nc * ns
    b_per_w = B // nw
    chunk = 128
    n_chunks = b_per_w // chunk
    mesh = plsc.VectorSubcoreMesh(core_axis_name="c", subcore_axis_name="s")
    kern = pl.kernel(
        functools.partial(_sc_gather_body, n_chunks=n_chunks, chunk=chunk,
                          b_per_w=b_per_w, nc=nc),
        out_type=jax.ShapeDtypeStruct((B, UDP), jnp.float32),
        mesh=mesh,
        scratch_types=[
            pltpu.VMEM((b_per_w,), jnp.int32),
            pltpu.VMEM((b_per_w, UDP), jnp.float32),
            pltpu.SemaphoreType.DMA,
        ],
        compiler_params=pltpu.CompilerParams(use_tc_tiling_on_sc=True),
    )
    return kern(jnp.pad(unit_table, ((0, 0), (0, UDP - UD))), unit_ids)


NSLOT = 20
KL = NSLOT * NT


def _ext_block(table, query, nrows, t):
    s = jnp.sum(table * query, axis=1, keepdims=True)
    m = jnp.max(s[:nrows, :])
    w = jnp.exp(s - m)
    z = lambda c: jnp.zeros((NT, c), jnp.float32)
    parts = [(SD * t, None), (0, table * w), (2 * SD - SD * t, None),
             (t, None), (0, w), (3 - t, None)]
    return jnp.concatenate(
        [p if p is not None else z(c) for c, p in parts if p is not None or c],
        axis=1)


def _tc_pool_body(uf_ref, aid_ref, tid_ref, sid_ref,
                  at_ref, tt_ref, st_ref, aq_ref, tq_ref, sq_ref, out_ref):
    R = uf_ref.shape[0]
    out_ref[:, 0:UD] = uf_ref[:, 0:UD]

    extall = jnp.concatenate([
        _ext_block(at_ref[...], aq_ref[...], 14, 0),
        _ext_block(tt_ref[...], tq_ref[...], 16, 1),
        _ext_block(st_ref[...], sq_ref[...], 4, 2)], axis=0)
    ri = lax.broadcasted_iota(jnp.int32, (KL, 48), 0)
    ci = lax.broadcasted_iota(jnp.int32, (KL, 48), 1)
    sel = (ci == ((ri // (8 * NT)).clip(0, 2) * NT + ri % NT))
    ext = jnp.dot(sel.astype(jnp.float32), extall,
                  preferred_element_type=jnp.float32,
                  precision=lax.Precision.HIGHEST)

    def spread_mat(nslots, slot0):
        sr = lax.broadcasted_iota(jnp.int32, (nslots, KL), 0)
        sc = lax.broadcasted_iota(jnp.int32, (nslots, KL), 1)
        return (sc // NT == sr + slot0).astype(jnp.float32)

    idr = (jnp.dot(aid_ref[...].astype(jnp.float32), spread_mat(8, 0),
                   preferred_element_type=jnp.float32)
           + jnp.dot(tid_ref[...].astype(jnp.float32), spread_mat(8, 8),
                     preferred_element_type=jnp.float32)
           + jnp.dot(sid_ref[...].astype(jnp.float32), spread_mat(4, 16),
                     preferred_element_type=jnp.float32))
    jmod = lax.broadcasted_iota(jnp.int32, (1, KL), 1) % NT
    eq = (idr == jmod.astype(jnp.float32)).astype(jnp.float32)
    nd = jnp.dot(eq, ext, preferred_element_type=jnp.float32)

    rec = 1.0 / nd[:, 96:99]
    rr = lax.broadcasted_iota(jnp.int32, (3, 3 * SD), 0)
    rc = lax.broadcasted_iota(jnp.int32, (3, 3 * SD), 1)
    rep3 = (rc // SD == rr).astype(jnp.float32)
    rec_hi = rec.astype(jnp.bfloat16).astype(jnp.float32)
    rec_lo = rec - rec_hi
    rec96 = (jnp.dot(rec_hi, rep3, preferred_element_type=jnp.float32)
             + jnp.dot(rec_lo, rep3, preferred_element_type=jnp.float32))
    out_ref[:, UD:OUT_D] = nd[:, 0:3 * SD] * rec96


def _pool_tc(ufeat, ability_ids, trait_ids, status_ids,
             at_p, tt_p, st_p, aq, tq, sq, *, interpret=False):
    R = 2048
    grid = (B // R,)
    row_spec = lambda w: pl.BlockSpec((R, w), lambda i: (i, 0))
    full = lambda a: pl.BlockSpec(a.shape, lambda i: (0, 0))
    return pl.pallas_call(
        _tc_pool_body,
        grid=grid,
        in_specs=[row_spec(UDP), row_spec(8), row_spec(8), row_spec(4),
                  full(at_p), full(tt_p), full(st_p),
                  full(aq), full(tq), full(sq)],
        out_specs=row_spec(OUT_D),
        out_shape=jax.ShapeDtypeStruct((B, OUT_D), jnp.float32),
        interpret=interpret,
    )(ufeat, ability_ids, trait_ids, status_ids,
      at_p, tt_p, st_p, aq, tq, sq)


def kernel(unit_ids, ability_ids, trait_ids, status_ids,
           unit_table, ability_table, trait_table, status_table,
           ability_query, trait_query, status_query):
    ufeat = _unit_gather_sc(unit_table, unit_ids)
    at_p = jnp.zeros((NT, SD), jnp.float32).at[:14].set(ability_table)
    tt_p = trait_table
    st_p = jnp.zeros((NT, SD), jnp.float32).at[:4].set(status_table)
    return _pool_tc(ufeat, ability_ids, trait_ids, status_ids,
                    at_p, tt_p, st_p,
                    ability_query.reshape(1, SD),
                    trait_query.reshape(1, SD),
                    status_query.reshape(1, SD))


# speedup vs baseline: 1.2795x; 1.0176x over previous
import functools

import jax
import jax.numpy as jnp
from jax import lax
from jax.experimental import pallas as pl
from jax.experimental.pallas import tpu as pltpu, tpu_sc as plsc

B = 16384
UD = 64
UDP = 128
SD = 32
NT = 16
OUT_D = UD + 3 * SD


def _sc_gather_body(table_hbm, idx_hbm, out_hbm, idx_v, rows_v, sem,
                    *, n_chunks, chunk, b_per_w, nc):
    wid = lax.axis_index("s") * nc + lax.axis_index("c")
    base = wid * b_per_w
    pltpu.sync_copy(idx_hbm.at[pl.ds(base, b_per_w)], idx_v)
    copies = [
        pltpu.async_copy(table_hbm.at[idx_v.at[pl.ds(j * chunk, chunk)]],
                         rows_v.at[pl.ds(j * chunk, chunk)], sem)
        for j in range(n_chunks)
    ]
    for c in copies:
        c.wait()
    pltpu.sync_copy(rows_v, out_hbm.at[pl.ds(base, b_per_w)])


def _unit_gather_sc(unit_table, unit_ids):
    info = plsc.get_sparse_core_info()
    nc, ns = info.num_cores, info.num_subcores
    nw = nc * ns
    b_per_w = B // nw
    chunk = 128
    n_chunks = b_per_w // chunk
    mesh = plsc.VectorSubcoreMesh(core_axis_name="c", subcore_axis_name="s")
    kern = pl.kernel(
        functools.partial(_sc_gather_body, n_chunks=n_chunks, chunk=chunk,
                          b_per_w=b_per_w, nc=nc),
        out_type=jax.ShapeDtypeStruct((B, UDP), jnp.float32),
        mesh=mesh,
        scratch_types=[
            pltpu.VMEM((b_per_w,), jnp.int32),
            pltpu.VMEM((b_per_w, UDP), jnp.float32),
            pltpu.SemaphoreType.DMA,
        ],
        compiler_params=pltpu.CompilerParams(use_tc_tiling_on_sc=True),
    )
    return kern(jnp.pad(unit_table, ((0, 0), (0, UDP - UD))), unit_ids)


NSLOT = 20
KL = NSLOT * NT


def _ext_block(table, query, nrows, t):
    s = jnp.sum(table * query, axis=1, keepdims=True)
    m = jnp.max(s[:nrows, :])
    w = jnp.exp(s - m)
    z = lambda c: jnp.zeros((NT, c), jnp.float32)
    parts = [(SD * t, None), (0, table * w), (2 * SD - SD * t, None),
             (t, None), (0, w), (3 - t, None)]
    return jnp.concatenate(
        [p if p is not None else z(c) for c, p in parts if p is not None or c],
        axis=1)


def _tc_pool_body(uf_ref, aid_ref, tid_ref, sid_ref,
                  at_ref, tt_ref, st_ref, aq_ref, tq_ref, sq_ref, out_ref):
    R = uf_ref.shape[0]
    out_ref[:, 0:UD] = uf_ref[:, 0:UD]

    extall = jnp.concatenate([
        _ext_block(at_ref[...], aq_ref[...], 14, 0),
        _ext_block(tt_ref[...], tq_ref[...], 16, 1),
        _ext_block(st_ref[...], sq_ref[...], 4, 2)], axis=0)
    ri = lax.broadcasted_iota(jnp.int32, (KL, 48), 0)
    ci = lax.broadcasted_iota(jnp.int32, (KL, 48), 1)
    sel = (ci == ((ri // (8 * NT)).clip(0, 2) * NT + ri % NT))
    ext = jnp.dot(sel.astype(jnp.float32), extall,
                  preferred_element_type=jnp.float32,
                  precision=lax.Precision.HIGHEST)

    def spread_mat(nslots, slot0):
        sr = lax.broadcasted_iota(jnp.int32, (nslots, KL), 0)
        sc = lax.broadcasted_iota(jnp.int32, (nslots, KL), 1)
        return (sc // NT == sr + slot0).astype(jnp.float32)

    idr = (jnp.dot(aid_ref[...].astype(jnp.float32), spread_mat(8, 0),
                   preferred_element_type=jnp.float32)
           + jnp.dot(tid_ref[...].astype(jnp.float32), spread_mat(8, 8),
                     preferred_element_type=jnp.float32)
           + jnp.dot(sid_ref[...].astype(jnp.float32), spread_mat(4, 16),
                     preferred_element_type=jnp.float32))
    jmod = lax.broadcasted_iota(jnp.int32, (1, KL), 1) % NT
    eq = (idr == jmod.astype(jnp.float32)).astype(jnp.float32)
    nd = jnp.dot(eq, ext, preferred_element_type=jnp.float32)

    rec = 1.0 / nd[:, 96:99]
    rr = lax.broadcasted_iota(jnp.int32, (3, 3 * SD), 0)
    rc = lax.broadcasted_iota(jnp.int32, (3, 3 * SD), 1)
    rep3 = (rc // SD == rr).astype(jnp.float32)
    rec_hi = rec.astype(jnp.bfloat16).astype(jnp.float32)
    rec_lo = rec - rec_hi
    rec96 = (jnp.dot(rec_hi, rep3, preferred_element_type=jnp.float32)
             + jnp.dot(rec_lo, rep3, preferred_element_type=jnp.float32))
    out_ref[:, UD:OUT_D] = nd[:, 0:3 * SD] * rec96


def _pool_tc(ufeat, ability_ids, trait_ids, status_ids,
             at_p, tt_p, st_p, aq, tq, sq, *, interpret=False):
    R = 4096
    grid = (B // R,)
    row_spec = lambda w: pl.BlockSpec((R, w), lambda i: (i, 0))
    full = lambda a: pl.BlockSpec(a.shape, lambda i: (0, 0))
    return pl.pallas_call(
        _tc_pool_body,
        grid=grid,
        in_specs=[row_spec(UDP), row_spec(8), row_spec(8), row_spec(4),
                  full(at_p), full(tt_p), full(st_p),
                  full(aq), full(tq), full(sq)],
        out_specs=row_spec(OUT_D),
        out_shape=jax.ShapeDtypeStruct((B, OUT_D), jnp.float32),
        interpret=interpret,
    )(ufeat, ability_ids, trait_ids, status_ids,
      at_p, tt_p, st_p, aq, tq, sq)


def kernel(unit_ids, ability_ids, trait_ids, status_ids,
           unit_table, ability_table, trait_table, status_table,
           ability_query, trait_query, status_query):
    ufeat = _unit_gather_sc(unit_table, unit_ids)
    at_p = jnp.zeros((NT, SD), jnp.float32).at[:14].set(ability_table)
    tt_p = trait_table
    st_p = jnp.zeros((NT, SD), jnp.float32).at[:4].set(status_table)
    return _pool_tc(ufeat, ability_ids, trait_ids, status_ids,
                    at_p, tt_p, st_p,
                    ability_query.reshape(1, SD),
                    trait_query.reshape(1, SD),
                    status_query.reshape(1, SD))


# speedup vs baseline: 1.2796x; 1.0002x over previous
import functools

import jax
import jax.numpy as jnp
from jax import lax
from jax.experimental import pallas as pl
from jax.experimental.pallas import tpu as pltpu, tpu_sc as plsc

B = 16384
UD = 64
UDP = 128
SD = 32
NT = 16
OUT_D = UD + 3 * SD


def _sc_gather_body(table_hbm, idx_hbm, out_hbm, idx_v, rows_v, sem,
                    *, n_chunks, chunk, b_per_w, nc):
    wid = lax.axis_index("s") * nc + lax.axis_index("c")
    base = wid * b_per_w
    pltpu.sync_copy(idx_hbm.at[pl.ds(base, b_per_w)], idx_v)
    copies = [
        pltpu.async_copy(table_hbm.at[idx_v.at[pl.ds(j * chunk, chunk)]],
                         rows_v.at[pl.ds(j * chunk, chunk)], sem)
        for j in range(n_chunks)
    ]
    for c in copies:
        c.wait()
    pltpu.sync_copy(rows_v, out_hbm.at[pl.ds(base, b_per_w)])


def _unit_gather_sc(unit_table, unit_ids):
    info = plsc.get_sparse_core_info()
    nc, ns = info.num_cores, info.num_subcores
    nw = nc * ns
    b_per_w = B // nw
    chunk = 128
    n_chunks = b_per_w // chunk
    mesh = plsc.VectorSubcoreMesh(core_axis_name="c", subcore_axis_name="s")
    kern = pl.kernel(
        functools.partial(_sc_gather_body, n_chunks=n_chunks, chunk=chunk,
                          b_per_w=b_per_w, nc=nc),
        out_type=jax.ShapeDtypeStruct((B, UDP), jnp.float32),
        mesh=mesh,
        scratch_types=[
            pltpu.VMEM((b_per_w,), jnp.int32),
            pltpu.VMEM((b_per_w, UDP), jnp.float32),
            pltpu.SemaphoreType.DMA,
        ],
        compiler_params=pltpu.CompilerParams(use_tc_tiling_on_sc=True),
    )
    return kern(jnp.pad(unit_table, ((0, 0), (0, UDP - UD))), unit_ids)


NSLOT = 20
KL = NSLOT * NT


def _ext_block(table, query, t):
    nrows = table.shape[0]
    s = jnp.sum(table * query, axis=1, keepdims=True)
    m = jnp.max(s)
    w = jnp.exp(s - m)
    z = lambda c: jnp.zeros((nrows, c), jnp.float32)
    parts = [(SD * t, None), (0, table * w), (2 * SD - SD * t, None),
             (t, None), (0, w), (3 - t, None)]
    ext = jnp.concatenate(
        [p if p is not None else z(c) for c, p in parts if p is not None or c],
        axis=1)
    if nrows < NT:
        ext = jnp.concatenate(
            [ext, jnp.zeros((NT - nrows, ext.shape[1]), jnp.float32)], axis=0)
    return ext


def _tc_pool_body(uf_ref, aid_ref, tid_ref, sid_ref,
                  at_ref, tt_ref, st_ref, aq_ref, tq_ref, sq_ref, out_ref):
    R = uf_ref.shape[0]
    out_ref[:, 0:UD] = uf_ref[:, 0:UD]

    extall = jnp.concatenate([
        _ext_block(at_ref[...], aq_ref[...], 0),
        _ext_block(tt_ref[...], tq_ref[...], 1),
        _ext_block(st_ref[...], sq_ref[...], 2)], axis=0)
    ri = lax.broadcasted_iota(jnp.int32, (KL, 48), 0)
    ci = lax.broadcasted_iota(jnp.int32, (KL, 48), 1)
    sel = (ci == ((ri // (8 * NT)).clip(0, 2) * NT + ri % NT))
    ext = jnp.dot(sel.astype(jnp.float32), extall,
                  preferred_element_type=jnp.float32,
                  precision=lax.Precision.HIGHEST)

    def spread_mat(nslots, slot0):
        sr = lax.broadcasted_iota(jnp.int32, (nslots, KL), 0)
        sc = lax.broadcasted_iota(jnp.int32, (nslots, KL), 1)
        return (sc // NT == sr + slot0).astype(jnp.float32)

    idr = (jnp.dot(aid_ref[...].astype(jnp.float32), spread_mat(8, 0),
                   preferred_element_type=jnp.float32)
           + jnp.dot(tid_ref[...].astype(jnp.float32), spread_mat(8, 8),
                     preferred_element_type=jnp.float32)
           + jnp.dot(sid_ref[...].astype(jnp.float32), spread_mat(4, 16),
                     preferred_element_type=jnp.float32))
    jmod = lax.broadcasted_iota(jnp.int32, (1, KL), 1) % NT
    eq = (idr == jmod.astype(jnp.float32)).astype(jnp.float32)
    nd = jnp.dot(eq, ext, preferred_element_type=jnp.float32)

    rec = 1.0 / nd[:, 96:99]
    rr = lax.broadcasted_iota(jnp.int32, (3, 3 * SD), 0)
    rc = lax.broadcasted_iota(jnp.int32, (3, 3 * SD), 1)
    rep3 = (rc // SD == rr).astype(jnp.float32)
    rec_hi = rec.astype(jnp.bfloat16).astype(jnp.float32)
    rec_lo = rec - rec_hi
    rec96 = (jnp.dot(rec_hi, rep3, preferred_element_type=jnp.float32)
             + jnp.dot(rec_lo, rep3, preferred_element_type=jnp.float32))
    out_ref[:, UD:OUT_D] = nd[:, 0:3 * SD] * rec96


def _pool_tc(ufeat, ability_ids, trait_ids, status_ids,
             at_p, tt_p, st_p, aq, tq, sq, *, interpret=False):
    R = 4096
    grid = (B // R,)
    row_spec = lambda w: pl.BlockSpec((R, w), lambda i: (i, 0))
    full = lambda a: pl.BlockSpec(a.shape, lambda i, _n=a.ndim: (0,) * _n)
    return pl.pallas_call(
        _tc_pool_body,
        grid=grid,
        in_specs=[row_spec(UDP), row_spec(8), row_spec(8), row_spec(4),
                  full(at_p), full(tt_p), full(st_p),
                  full(aq), full(tq), full(sq)],
        out_specs=row_spec(OUT_D),
        out_shape=jax.ShapeDtypeStruct((B, OUT_D), jnp.float32),
        interpret=interpret,
    )(ufeat, ability_ids, trait_ids, status_ids,
      at_p, tt_p, st_p, aq, tq, sq)


def kernel(unit_ids, ability_ids, trait_ids, status_ids,
           unit_table, ability_table, trait_table, status_table,
           ability_query, trait_query, status_query):
    ufeat = _unit_gather_sc(unit_table, unit_ids)
    return _pool_tc(ufeat, ability_ids, trait_ids, status_ids,
                    ability_table, trait_table, status_table,
                    ability_query, trait_query, status_query)


# speedup vs baseline: 1.3375x; 1.0452x over previous
import functools

import jax
import jax.numpy as jnp
from jax import lax
from jax.experimental import pallas as pl
from jax.experimental.pallas import tpu as pltpu, tpu_sc as plsc

B = 16384
UD = 64
UDP = 128
SD = 32
NT = 16
OUT_D = UD + 3 * SD


def _sc_gather_body(table_hbm, idx_hbm, out_hbm, idx_v, rows_v, sem,
                    *, n_chunks, chunk, b_per_w, nc):
    wid = lax.axis_index("s") * nc + lax.axis_index("c")
    base = wid * b_per_w
    pltpu.sync_copy(idx_hbm.at[pl.ds(base, b_per_w)], idx_v)
    copies = [
        pltpu.async_copy(table_hbm.at[idx_v.at[pl.ds(j * chunk, chunk)]],
                         rows_v.at[pl.ds(j * chunk, chunk)], sem)
        for j in range(n_chunks)
    ]
    for c in copies:
        c.wait()
    pltpu.sync_copy(rows_v, out_hbm.at[pl.ds(base, b_per_w)])


def _unit_gather_sc(unit_table, unit_ids):
    info = plsc.get_sparse_core_info()
    nc, ns = info.num_cores, info.num_subcores
    nw = nc * ns
    b_per_w = B // nw
    chunk = 128
    n_chunks = b_per_w // chunk
    mesh = plsc.VectorSubcoreMesh(core_axis_name="c", subcore_axis_name="s")
    kern = pl.kernel(
        functools.partial(_sc_gather_body, n_chunks=n_chunks, chunk=chunk,
                          b_per_w=b_per_w, nc=nc),
        out_type=jax.ShapeDtypeStruct((B, UDP), jnp.float32),
        mesh=mesh,
        scratch_types=[
            pltpu.VMEM((b_per_w,), jnp.int32),
            pltpu.VMEM((b_per_w, UDP), jnp.float32),
            pltpu.SemaphoreType.DMA,
        ],
        compiler_params=pltpu.CompilerParams(use_tc_tiling_on_sc=True),
    )
    return kern(jnp.pad(unit_table, ((0, 0), (0, UDP - UD))), unit_ids)


NSLOT = 20
KL = 256
AW, TW, SW = 14, 16, 4
TBASE, SBASE = 8 * AW, 8 * AW + 8 * TW


def _ext_block(table, query, t):
    nrows = table.shape[0]
    s = jnp.sum(table * query, axis=1, keepdims=True)
    m = jnp.max(s)
    w = jnp.exp(s - m)
    z = lambda c: jnp.zeros((nrows, c), jnp.float32)
    parts = [(SD * t, None), (0, table * w), (2 * SD - SD * t, None),
             (t, None), (0, w), (3 - t, None)]
    ext = jnp.concatenate(
        [p if p is not None else z(c) for c, p in parts if p is not None or c],
        axis=1)
    if nrows < NT:
        ext = jnp.concatenate(
            [ext, jnp.zeros((NT - nrows, ext.shape[1]), jnp.float32)], axis=0)
    return ext


def _tc_pool_body(uf_ref, aid_ref, tid_ref, sid_ref,
                  at_ref, tt_ref, st_ref, aq_ref, tq_ref, sq_ref, out_ref):
    R = uf_ref.shape[0]
    out_ref[:, 0:UD] = uf_ref[:, 0:UD]

    extall = jnp.concatenate([
        _ext_block(at_ref[...], aq_ref[...], 0),
        _ext_block(tt_ref[...], tq_ref[...], 1),
        _ext_block(st_ref[...], sq_ref[...], 2)], axis=0)
    k1 = lax.broadcasted_iota(jnp.int32, (KL, 48), 0)
    tno1 = jnp.where(k1 < TBASE, 0, jnp.where(k1 < SBASE, 1, 2))
    jm1 = jnp.where(k1 < TBASE, k1 % AW,
                    jnp.where(k1 < SBASE, (k1 - TBASE) % TW,
                              (k1 - SBASE) % SW))
    ci = lax.broadcasted_iota(jnp.int32, (KL, 48), 1)
    sel = (ci == tno1 * NT + jm1)
    ext = jnp.dot(sel.astype(jnp.float32), extall,
                  preferred_element_type=jnp.float32,
                  precision=lax.Precision.HIGHEST)

    def spread_mat(nslots, width, base):
        sr = lax.broadcasted_iota(jnp.int32, (nslots, KL), 0)
        sc = lax.broadcasted_iota(jnp.int32, (nslots, KL), 1)
        ink = (sc >= base) & (sc < base + nslots * width)
        return (ink & ((sc - base) // width == sr)).astype(jnp.float32)

    idr = (jnp.dot(aid_ref[...].astype(jnp.float32), spread_mat(8, AW, 0),
                   preferred_element_type=jnp.float32)
           + jnp.dot(tid_ref[...].astype(jnp.float32),
                     spread_mat(8, TW, TBASE),
                     preferred_element_type=jnp.float32)
           + jnp.dot(sid_ref[...].astype(jnp.float32),
                     spread_mat(4, SW, SBASE),
                     preferred_element_type=jnp.float32))
    k2 = lax.broadcasted_iota(jnp.int32, (1, KL), 1)
    jmod = jnp.where(k2 < TBASE, k2 % AW,
                     jnp.where(k2 < SBASE, (k2 - TBASE) % TW,
                               (k2 - SBASE) % SW))
    eq = (idr == jmod.astype(jnp.float32)).astype(jnp.float32)
    nd = jnp.dot(eq, ext, preferred_element_type=jnp.float32)

    rec = 1.0 / nd[:, 96:99]
    rr = lax.broadcasted_iota(jnp.int32, (3, 3 * SD), 0)
    rc = lax.broadcasted_iota(jnp.int32, (3, 3 * SD), 1)
    rep3 = (rc // SD == rr).astype(jnp.float32)
    rec_hi = rec.astype(jnp.bfloat16).astype(jnp.float32)
    rec_lo = rec - rec_hi
    rec96 = (jnp.dot(rec_hi, rep3, preferred_element_type=jnp.float32)
             + jnp.dot(rec_lo, rep3, preferred_element_type=jnp.float32))
    out_ref[:, UD:OUT_D] = nd[:, 0:3 * SD] * rec96


def _pool_tc(ufeat, ability_ids, trait_ids, status_ids,
             at_p, tt_p, st_p, aq, tq, sq, *, interpret=False):
    R = 4096
    grid = (B // R,)
    row_spec = lambda w: pl.BlockSpec((R, w), lambda i: (i, 0))
    full = lambda a: pl.BlockSpec(a.shape, lambda i, _n=a.ndim: (0,) * _n)
    return pl.pallas_call(
        _tc_pool_body,
        grid=grid,
        in_specs=[row_spec(UDP), row_spec(8), row_spec(8), row_spec(4),
                  full(at_p), full(tt_p), full(st_p),
                  full(aq), full(tq), full(sq)],
        out_specs=row_spec(OUT_D),
        out_shape=jax.ShapeDtypeStruct((B, OUT_D), jnp.float32),
        interpret=interpret,
    )(ufeat, ability_ids, trait_ids, status_ids,
      at_p, tt_p, st_p, aq, tq, sq)


def kernel(unit_ids, ability_ids, trait_ids, status_ids,
           unit_table, ability_table, trait_table, status_table,
           ability_query, trait_query, status_query):
    ufeat = _unit_gather_sc(unit_table, unit_ids)
    return _pool_tc(ufeat, ability_ids, trait_ids, status_ids,
                    ability_table, trait_table, status_table,
                    ability_query, trait_query, status_query)


# speedup vs baseline: 1.3418x; 1.0032x over previous
import functools

import jax
import jax.numpy as jnp
from jax import lax
from jax.experimental import pallas as pl
from jax.experimental.pallas import tpu as pltpu, tpu_sc as plsc

B = 16384
UD = 64
UDP = 128
SD = 32
NT = 16
OUT_D = UD + 3 * SD


def _sc_gather_body(table_hbm, idx_hbm, out_hbm, idx_v, rows_v, sem,
                    *, n_chunks, chunk, b_per_w, nc):
    wid = lax.axis_index("s") * nc + lax.axis_index("c")
    base = wid * b_per_w
    pltpu.sync_copy(idx_hbm.at[pl.ds(base, b_per_w)], idx_v)
    copies = [
        pltpu.async_copy(table_hbm.at[idx_v.at[pl.ds(j * chunk, chunk)]],
                         rows_v.at[pl.ds(j * chunk, chunk)], sem)
        for j in range(n_chunks)
    ]
    for c in copies:
        c.wait()
    pltpu.sync_copy(rows_v, out_hbm.at[pl.ds(base, b_per_w)])


def _unit_gather_sc(unit_table, unit_ids):
    info = plsc.get_sparse_core_info()
    nc, ns = info.num_cores, info.num_subcores
    nw = nc * ns
    b_per_w = B // nw
    chunk = 128
    n_chunks = b_per_w // chunk
    mesh = plsc.VectorSubcoreMesh(core_axis_name="c", subcore_axis_name="s")
    kern = pl.kernel(
        functools.partial(_sc_gather_body, n_chunks=n_chunks, chunk=chunk,
                          b_per_w=b_per_w, nc=nc),
        out_type=jax.ShapeDtypeStruct((B, UDP), jnp.float32),
        mesh=mesh,
        scratch_types=[
            pltpu.VMEM((b_per_w,), jnp.int32),
            pltpu.VMEM((b_per_w, UDP), jnp.float32),
            pltpu.SemaphoreType.DMA,
        ],
        compiler_params=pltpu.CompilerParams(use_tc_tiling_on_sc=True),
    )
    return kern(jnp.pad(unit_table, ((0, 0), (0, UDP - UD))), unit_ids)


NSLOT = 20
KL = 256
AW, TW, SW = 14, 16, 4
TBASE, SBASE = 8 * AW, 8 * AW + 8 * TW


def _ext_block(table, query, t):
    nrows = table.shape[0]
    s = jnp.sum(table * query, axis=1, keepdims=True)
    m = jnp.max(s)
    w = jnp.exp(s - m)
    z = lambda c: jnp.zeros((nrows, c), jnp.float32)
    parts = [(SD * t, None), (0, table * w), (2 * SD - SD * t, None),
             (t, None), (0, w), (3 - t, None)]
    ext = jnp.concatenate(
        [p if p is not None else z(c) for c, p in parts if p is not None or c],
        axis=1)
    if nrows < NT:
        ext = jnp.concatenate(
            [ext, jnp.zeros((NT - nrows, ext.shape[1]), jnp.float32)], axis=0)
    return ext


def _tc_pool_body(uf_ref, aid_ref, tid_ref, sid_ref,
                  at_ref, tt_ref, st_ref, aq_ref, tq_ref, sq_ref, out_ref):
    R = uf_ref.shape[0]
    out_ref[:, 0:UD] = uf_ref[:, 0:UD]

    extall = jnp.concatenate([
        _ext_block(at_ref[...], aq_ref[...], 0),
        _ext_block(tt_ref[...], tq_ref[...], 1),
        _ext_block(st_ref[...], sq_ref[...], 2)], axis=0)
    k1 = lax.broadcasted_iota(jnp.int32, (KL, 48), 0)
    tno1 = jnp.where(k1 < TBASE, 0, jnp.where(k1 < SBASE, 1, 2))
    jm1 = jnp.where(k1 < TBASE, k1 % AW,
                    jnp.where(k1 < SBASE, (k1 - TBASE) % TW,
                              (k1 - SBASE) % SW))
    ci = lax.broadcasted_iota(jnp.int32, (KL, 48), 1)
    sel = (ci == tno1 * NT + jm1)
    ext = jnp.dot(sel.astype(jnp.float32), extall,
                  preferred_element_type=jnp.float32,
                  precision=lax.Precision.HIGHEST)

    def spread_mat(nslots, width, base):
        sr = lax.broadcasted_iota(jnp.int32, (nslots, KL), 0)
        sc = lax.broadcasted_iota(jnp.int32, (nslots, KL), 1)
        ink = (sc >= base) & (sc < base + nslots * width)
        return (ink & ((sc - base) // width == sr)).astype(jnp.float32)

    idr = (jnp.dot(aid_ref[...].astype(jnp.float32), spread_mat(8, AW, 0),
                   preferred_element_type=jnp.float32)
           + jnp.dot(tid_ref[...].astype(jnp.float32),
                     spread_mat(8, TW, TBASE),
                     preferred_element_type=jnp.float32)
           + jnp.dot(sid_ref[...].astype(jnp.float32),
                     spread_mat(4, SW, SBASE),
                     preferred_element_type=jnp.float32))
    k2 = lax.broadcasted_iota(jnp.int32, (1, KL), 1)
    jmod = jnp.where(k2 < TBASE, k2 % AW,
                     jnp.where(k2 < SBASE, (k2 - TBASE) % TW,
                               (k2 - SBASE) % SW))
    eq = (idr == jmod.astype(jnp.float32)).astype(jnp.float32)
    nd = jnp.dot(eq, ext, preferred_element_type=jnp.float32)

    rec = 1.0 / nd[:, 96:99]
    rr = lax.broadcasted_iota(jnp.int32, (3, 3 * SD), 0)
    rc = lax.broadcasted_iota(jnp.int32, (3, 3 * SD), 1)
    rep3 = (rc // SD == rr).astype(jnp.float32)
    rec_hi = rec.astype(jnp.bfloat16).astype(jnp.float32)
    rec_lo = rec - rec_hi
    rec96 = (jnp.dot(rec_hi, rep3, preferred_element_type=jnp.float32)
             + jnp.dot(rec_lo, rep3, preferred_element_type=jnp.float32))
    out_ref[:, UD:OUT_D] = nd[:, 0:3 * SD] * rec96


def _pool_tc(ufeat, ability_ids, trait_ids, status_ids,
             at_p, tt_p, st_p, aq, tq, sq):
    R = 4096
    grid = (B // R,)
    row_spec = lambda w: pl.BlockSpec((R, w), lambda i: (i, 0))
    full = lambda a: pl.BlockSpec(a.shape, lambda i, _n=a.ndim: (0,) * _n)
    return pl.pallas_call(
        _tc_pool_body,
        grid=grid,
        in_specs=[row_spec(UDP), row_spec(8), row_spec(8), row_spec(4),
                  full(at_p), full(tt_p), full(st_p),
                  full(aq), full(tq), full(sq)],
        out_specs=row_spec(OUT_D),
        out_shape=jax.ShapeDtypeStruct((B, OUT_D), jnp.float32),
    )(ufeat, ability_ids, trait_ids, status_ids,
      at_p, tt_p, st_p, aq, tq, sq)


def kernel(unit_ids, ability_ids, trait_ids, status_ids,
           unit_table, ability_table, trait_table, status_table,
           ability_query, trait_query, status_query):
    ufeat = _unit_gather_sc(unit_table, unit_ids)
    return _pool_tc(ufeat, ability_ids, trait_ids, status_ids,
                    ability_table, trait_table, status_table,
                    ability_query, trait_query, status_query)
